# baseline (speedup 1.0000x reference)
import functools
import jax
import jax.numpy as jnp
from jax import lax
from jax.experimental import pallas as pl
from jax.experimental.pallas import tpu as pltpu
from jax.experimental.pallas import tpu_sc as plsc

N_ENT = 1000000
N_REL = 1000
D = 64
HD = 32
B = 16384
EPS = 1e-12
W = 128
NBLK = N_ENT // 8

NC = 2
NS = 16
L = 16
NW = NC * NS
BPW = B // NW
CH = 16
NCH = BPW // CH
GPC = CH // L
NBUF = 3


def _trig_body(ph_ref, cs_ref):
    ph = ph_ref[...]
    z = jnp.zeros_like(ph)
    cs_ref[...] = jnp.concatenate([jnp.cos(ph), jnp.sin(ph), z, z], axis=1)


def _make_cs(phases):
    return pl.pallas_call(
        _trig_body,
        out_shape=jax.ShapeDtypeStruct((N_REL, W), jnp.float32),
    )(phases)


def _fast_sqrt(x):
    y = lax.bitcast_convert_type(x, jnp.int32)
    y = jnp.int32(0x5F3759DF) - lax.shift_right_logical(y, 1)
    y = lax.bitcast_convert_type(y, jnp.float32)
    for _ in range(3):
        y = y * (1.5 - 0.5 * x * y * y)
    return x * y


_MESH = plsc.VectorSubcoreMesh(core_axis_name="c", subcore_axis_name="s")


@functools.partial(
    pl.kernel,
    mesh=_MESH,
    compiler_params=pltpu.CompilerParams(
        needs_layout_passes=False, use_tc_tiling_on_sc=True
    ),
    out_type=jax.ShapeDtypeStruct((B,), jnp.float32),
    scratch_types=[
        pltpu.VMEM((BPW + L,), jnp.int32),
        pltpu.VMEM((BPW + L,), jnp.int32),
        pltpu.VMEM((NCH, CH), jnp.int32),
        pltpu.VMEM((NBUF, CH, 8, D), jnp.float32),
        pltpu.VMEM((NBUF, CH, 8, D), jnp.float32),
        pltpu.VMEM((NBUF, CH, W), jnp.float32),
        pltpu.VMEM((BPW,), jnp.float32),
        pltpu.SemaphoreType.DMA((NBUF,)),
        pltpu.SemaphoreType.DMA((NBUF,)),
    ],
)
def _sc_kernel(h_hbm, t_hbm, r_hbm, ent_hbm, cs_hbm, out_hbm,
               hv, tv, rv, hstage, tstage, csstage, out_v,
               sems, csems):
    wid = lax.axis_index("s") * NC + lax.axis_index("c")

    pltpu.sync_copy(h_hbm.at[wid], hv.at[pl.ds(0, BPW)])
    pltpu.sync_copy(t_hbm.at[wid], tv.at[pl.ds(0, BPW)])
    pltpu.sync_copy(r_hbm.at[wid], rv)

    def fire(c, bi):
        hblk = lax.shift_right_logical(hv[pl.ds(c * CH, L)], 3)
        tblk = lax.shift_right_logical(tv[pl.ds(c * CH, L)], 3)
        for b in range(CH):
            pltpu.async_copy(
                ent_hbm.at[hblk[b]], hstage.at[bi, b], sems.at[bi]
            )
            pltpu.async_copy(
                ent_hbm.at[tblk[b]], tstage.at[bi, b], sems.at[bi]
            )
        pltpu.async_copy(cs_hbm.at[rv.at[c]], csstage.at[bi], csems.at[bi])

    for c0 in range(NBUF):
        fire(jnp.int32(c0), jnp.int32(c0))

    lane = lax.iota(jnp.int32, L)
    perms = [
        jnp.bitwise_and(lane + sh, L - 1).astype(jnp.int32) for sh in (8, 4, 2, 1)
    ]

    def chunk_body(c, carry):
        bi = lax.rem(c, jnp.int32(NBUF))
        pltpu.make_async_copy(
            ent_hbm.at[pl.ds(0, CH)], hstage.at[bi], sems.at[bi]
        ).wait()
        pltpu.make_async_copy(
            ent_hbm.at[pl.ds(0, CH)], tstage.at[bi], sems.at[bi]
        ).wait()
        pltpu.make_async_copy(
            cs_hbm.at[pl.ds(0, CH)], csstage.at[bi], csems.at[bi]
        ).wait()

        def body(g, carry2):
            hsr = jnp.bitwise_and(hv[pl.ds(c * CH + g * L, L)], 7)
            tsr = jnp.bitwise_and(tv[pl.ds(c * CH + g * L, L)], 7)
            res = jnp.zeros((L,), jnp.float32)
            for e in range(L):
                b = g * L + e
                acc = jnp.zeros((L,), jnp.float32)
                for k in range(HD // L):
                    hre = hstage[bi, b, hsr[e], pl.ds(k * L, L)]
                    him = hstage[bi, b, hsr[e], pl.ds(HD + k * L, L)]
                    tre = tstage[bi, b, tsr[e], pl.ds(k * L, L)]
                    tim = tstage[bi, b, tsr[e], pl.ds(HD + k * L, L)]
                    cc = csstage[bi, b, pl.ds(k * L, L)]
                    ss = csstage[bi, b, pl.ds(HD + k * L, L)]
                    dre = hre * cc - him * ss - tre
                    dim = hre * ss + him * cc - tim
                    acc = acc + _fast_sqrt(dre * dre + dim * dim + EPS)
                for p in perms:
                    acc = acc + jnp.take_along_axis(acc, p, axis=0)
                res = jnp.where(lane == e, acc, res)
            out_v[pl.ds((c * GPC + g) * L, L)] = jnp.zeros((L,), jnp.float32) - res
            return carry2

        lax.fori_loop(0, GPC, body, 0)

        @pl.when(c + NBUF < NCH)
        def _():
            fire(c + NBUF, bi)

        return carry

    lax.fori_loop(0, NCH, chunk_body, 0)

    pltpu.sync_copy(out_v, out_hbm.at[pl.ds(wid * BPW, BPW)])


def kernel(h, r, t, entity_embed, relation_phases):
    cs = _make_cs(relation_phases)
    ent3 = entity_embed.reshape(NBLK, 8, D)
    h2 = h.astype(jnp.int32).reshape(NW, BPW)
    t2 = t.astype(jnp.int32).reshape(NW, BPW)
    r3 = r.astype(jnp.int32).reshape(NW, NCH, CH)
    return _sc_kernel(h2, t2, r3, ent3, cs)


# speedup vs baseline: 1.0105x; 1.0105x over previous
import functools
import jax
import jax.numpy as jnp
from jax import lax
from jax.experimental import pallas as pl
from jax.experimental.pallas import tpu as pltpu
from jax.experimental.pallas import tpu_sc as plsc

N_ENT = 1000000
N_REL = 1000
D = 64
HD = 32
B = 16384
EPS = 1e-12
W = 128
NBLK = N_ENT // 8

NC = 2
NS = 16
L = 16
NW = NC * NS
BPW = B // NW
CH = 16
NCH = BPW // CH
GPC = CH // L
NBUF = 3


def _trig_body(ph_ref, cs_ref):
    ph = ph_ref[...]
    z = jnp.zeros_like(ph)
    cs_ref[...] = jnp.concatenate([jnp.cos(ph), jnp.sin(ph), z, z], axis=1)


def _make_cs(phases):
    return pl.pallas_call(
        _trig_body,
        out_shape=jax.ShapeDtypeStruct((N_REL, W), jnp.float32),
    )(phases)


def _fast_sqrt(x):
    y = lax.bitcast_convert_type(x, jnp.int32)
    y = jnp.int32(0x5F3759DF) - lax.shift_right_logical(y, 1)
    y = lax.bitcast_convert_type(y, jnp.float32)
    for _ in range(3):
        y = y * (1.5 - 0.5 * x * y * y)
    return x * y


_MESH = plsc.VectorSubcoreMesh(core_axis_name="c", subcore_axis_name="s")


@functools.partial(
    pl.kernel,
    mesh=_MESH,
    compiler_params=pltpu.CompilerParams(
        needs_layout_passes=False, use_tc_tiling_on_sc=True
    ),
    out_type=jax.ShapeDtypeStruct((B,), jnp.float32),
    scratch_types=[
        pltpu.VMEM((BPW + L,), jnp.int32),
        pltpu.VMEM((BPW + L,), jnp.int32),
        pltpu.VMEM((NCH, CH), jnp.int32),
        pltpu.VMEM((NBUF, CH, 8, D), jnp.float32),
        pltpu.VMEM((NBUF, CH, 8, D), jnp.float32),
        pltpu.VMEM((NBUF, CH, W), jnp.float32),
        pltpu.VMEM((BPW,), jnp.float32),
        pltpu.SemaphoreType.DMA((NBUF,)),
        pltpu.SemaphoreType.DMA((NBUF,)),
    ],
)
def _sc_kernel(h_hbm, t_hbm, r_hbm, ent_hbm, cs_hbm, out_hbm,
               hv, tv, rv, hstage, tstage, csstage, out_v,
               sems, csems):
    wid = lax.axis_index("s") * NC + lax.axis_index("c")

    pltpu.sync_copy(h_hbm.at[wid], hv.at[pl.ds(0, BPW)])
    pltpu.sync_copy(t_hbm.at[wid], tv.at[pl.ds(0, BPW)])
    pltpu.sync_copy(r_hbm.at[wid], rv)

    def fire(c, bi):
        def fbody(b, carry):
            hi = hv[pl.ds(c * CH + b, L)][0]
            ti = tv[pl.ds(c * CH + b, L)][0]
            pltpu.async_copy(
                ent_hbm.at[lax.shift_right_logical(hi, 3)],
                hstage.at[bi, b],
                sems.at[bi],
            )
            pltpu.async_copy(
                ent_hbm.at[lax.shift_right_logical(ti, 3)],
                tstage.at[bi, b],
                sems.at[bi],
            )
            return carry

        lax.fori_loop(0, CH, fbody, 0)
        pltpu.async_copy(cs_hbm.at[rv.at[c]], csstage.at[bi], csems.at[bi])

    for c0 in range(NBUF):
        fire(jnp.int32(c0), jnp.int32(c0))

    lane = lax.iota(jnp.int32, L)
    perms = [
        jnp.bitwise_and(lane + sh, L - 1).astype(jnp.int32) for sh in (8, 4, 2, 1)
    ]

    def chunk_body(c, carry):
        bi = lax.rem(c, jnp.int32(NBUF))
        pltpu.make_async_copy(
            ent_hbm.at[pl.ds(0, CH)], hstage.at[bi], sems.at[bi]
        ).wait()
        pltpu.make_async_copy(
            ent_hbm.at[pl.ds(0, CH)], tstage.at[bi], sems.at[bi]
        ).wait()
        pltpu.make_async_copy(
            cs_hbm.at[pl.ds(0, CH)], csstage.at[bi], csems.at[bi]
        ).wait()

        def body(g, carry2):
            hsr = jnp.bitwise_and(hv[pl.ds(c * CH + g * L, L)], 7)
            tsr = jnp.bitwise_and(tv[pl.ds(c * CH + g * L, L)], 7)
            res = jnp.zeros((L,), jnp.float32)
            for e in range(L):
                b = g * L + e
                acc = jnp.zeros((L,), jnp.float32)
                for k in range(HD // L):
                    hre = hstage[bi, b, hsr[e], pl.ds(k * L, L)]
                    him = hstage[bi, b, hsr[e], pl.ds(HD + k * L, L)]
                    tre = tstage[bi, b, tsr[e], pl.ds(k * L, L)]
                    tim = tstage[bi, b, tsr[e], pl.ds(HD + k * L, L)]
                    cc = csstage[bi, b, pl.ds(k * L, L)]
                    ss = csstage[bi, b, pl.ds(HD + k * L, L)]
                    dre = hre * cc - him * ss - tre
                    dim = hre * ss + him * cc - tim
                    acc = acc + _fast_sqrt(dre * dre + dim * dim + EPS)
                for p in perms:
                    acc = acc + jnp.take_along_axis(acc, p, axis=0)
                res = jnp.where(lane == e, acc, res)
            out_v[pl.ds((c * GPC + g) * L, L)] = jnp.zeros((L,), jnp.float32) - res
            return carry2

        lax.fori_loop(0, GPC, body, 0)

        @pl.when(c + NBUF < NCH)
        def _():
            fire(c + NBUF, bi)

        return carry

    lax.fori_loop(0, NCH, chunk_body, 0)

    pltpu.sync_copy(out_v, out_hbm.at[pl.ds(wid * BPW, BPW)])


def kernel(h, r, t, entity_embed, relation_phases):
    cs = _make_cs(relation_phases)
    ent3 = entity_embed.reshape(NBLK, 8, D)
    h2 = h.astype(jnp.int32).reshape(NW, BPW)
    t2 = t.astype(jnp.int32).reshape(NW, BPW)
    r3 = r.astype(jnp.int32).reshape(NW, NCH, CH)
    return _sc_kernel(h2, t2, r3, ent3, cs)
